# baseline (speedup 1.0000x reference)
import functools

import jax
import jax.numpy as jnp
from jax import lax
from jax.experimental import pallas as pl
from jax.experimental.pallas import tpu as pltpu
from jax.experimental.pallas import tpu_sc as plsc

N_NODES = 10000
D_FEAT = 128
N_EDGES = 320000
DEPTH = 3

NC = 2
NS = 16
NW = NC * NS
CHUNK = 64
NCH = 159
EPT = CHUNK * NCH
E_PAD = EPT * NW
TRIPLES = NCH // 3
RPT = 632
RPT_LAST = N_NODES - (NS - 1) * RPT

_mesh = plsc.VectorSubcoreMesh(
    core_axis_name="c", subcore_axis_name="s", num_cores=NC, num_subcores=NS
)


@functools.partial(
    pl.kernel,
    out_type=jax.ShapeDtypeStruct((NC, N_NODES, D_FEAT), jnp.float32),
    mesh=_mesh,
    scratch_types=[
        pltpu.VMEM_SHARED((N_NODES, D_FEAT), jnp.float32),
        pltpu.VMEM((NCH, CHUNK), jnp.int32),
        pltpu.VMEM((NCH, CHUNK), jnp.int32),
        pltpu.VMEM((CHUNK,), jnp.float32),
        pltpu.VMEM((CHUNK,), jnp.float32),
        pltpu.VMEM((CHUNK,), jnp.float32),
        pltpu.VMEM((CHUNK, D_FEAT), jnp.float32),
        pltpu.VMEM((CHUNK, D_FEAT), jnp.float32),
        pltpu.VMEM((CHUNK, D_FEAT), jnp.float32),
        pltpu.SemaphoreType.DMA,
        pltpu.SemaphoreType.DMA,
        pltpu.SemaphoreType.DMA,
        pltpu.SemaphoreType.DMA,
        pltpu.SemaphoreType.DMA,
        pltpu.SemaphoreType.DMA,
        pltpu.SemaphoreType.DMA,
        pltpu.SemaphoreType.DMA,
        pltpu.SemaphoreType.DMA,
    ],
    compiler_params=pltpu.CompilerParams(use_tc_tiling_on_sc=False),
)
def _spmm_partial(x_hbm, src_hbm, dst_hbm, w_hbm, zeros_hbm, part_hbm,
                  acc, src_all, dst_all, wb0, wb1, wb2, rb0, rb1, rb2,
                  sw0, sw1, sw2, sg0, sg1, sg2, ss0, ss1, ss2):
    c = lax.axis_index("c")
    s = lax.axis_index("s")
    wid = c * NS + s
    rbase = pl.multiple_of(s * RPT, 8)

    @pl.when(s < NS - 1)
    def _zero_main():
        pltpu.sync_copy(zeros_hbm.at[pl.ds(rbase, RPT)], acc.at[pl.ds(rbase, RPT)])

    @pl.when(s == NS - 1)
    def _zero_last():
        pltpu.sync_copy(zeros_hbm.at[pl.ds(rbase, RPT_LAST)],
                        acc.at[pl.ds(rbase, RPT_LAST)])

    pltpu.sync_copy(src_hbm.at[wid], src_all)
    pltpu.sync_copy(dst_hbm.at[wid], dst_all)

    plsc.subcore_barrier()

    wbufs = (wb0, wb1, wb2)
    rows_bufs = (rb0, rb1, rb2)
    sw = (sw0, sw1, sw2)
    sg = (sg0, sg1, sg2)
    ss = (ss0, ss1, ss2)

    def start_fetch(k, b):
        pltpu.async_copy(w_hbm.at[wid, k], wbufs[b], sw[b])
        pltpu.async_copy(x_hbm.at[src_all.at[k]], rows_bufs[b], sg[b])

    def wait_fetch(k, b):
        pltpu.make_async_copy(w_hbm.at[wid, k], wbufs[b], sw[b]).wait()
        pltpu.make_async_copy(x_hbm.at[src_all.at[k]], rows_bufs[b], sg[b]).wait()

    def mul_rows(k, b):
        rowsb = rows_bufs[b]

        def group(g, carry):
            wv = wbufs[b][pl.ds(g * 16, 16)]
            for e in range(16):
                row = g * 16 + e
                wgt = wv[e]
                for j in range(D_FEAT // 16):
                    sl = pl.ds(j * 16, 16)
                    rowsb[row, sl] = rowsb[row, sl] * wgt
            return carry

        lax.fori_loop(0, CHUNK // 16, group, 0)

    def start_scatter(k, b):
        pltpu.async_copy(rows_bufs[b], acc.at[dst_all.at[k]], ss[b], add=True)

    def wait_scatter(k, b):
        pltpu.make_async_copy(rows_bufs[b], acc.at[dst_all.at[k]], ss[b]).wait()

    start_fetch(0, 0)
    start_fetch(1, 1)

    def triple_body(t, carry):
        for u in range(3):
            k = 3 * t + u
            wait_fetch(k, u)
            mul_rows(k, u)
            start_scatter(k, u)
            b2 = (u + 2) % 3
            if u == 0:
                @pl.when(t > 0)
                def _refill0():
                    wait_scatter(k - 1, b2)
                    start_fetch(k + 2, b2)

                @pl.when(t == 0)
                def _first_fill():
                    start_fetch(k + 2, b2)
            else:
                @pl.when(t < TRIPLES - 1)
                def _refill():
                    wait_scatter(k - 1, b2)
                    start_fetch(k + 2, b2)
        return carry

    lax.fori_loop(0, TRIPLES, triple_body, 0)

    wait_scatter(NCH - 3, (NCH - 3) % 3)
    wait_scatter(NCH - 2, (NCH - 2) % 3)
    wait_scatter(NCH - 1, (NCH - 1) % 3)

    plsc.subcore_barrier()

    @pl.when(s < NS - 1)
    def _out_main():
        pltpu.sync_copy(acc.at[pl.ds(rbase, RPT)],
                        part_hbm.at[c, pl.ds(rbase, RPT)])

    @pl.when(s == NS - 1)
    def _out_last():
        pltpu.sync_copy(acc.at[pl.ds(rbase, RPT_LAST)],
                        part_hbm.at[c, pl.ds(rbase, RPT_LAST)])


def _combine_body(alpha_ref, p_ref, o_ref):
    o_ref[...] = alpha_ref[0] * (p_ref[0] + p_ref[1])


_BLK = 1000


def _combine(parts, alpha):
    alpha_arr = jnp.reshape(alpha, (1,)).astype(jnp.float32)
    return pl.pallas_call(
        _combine_body,
        out_shape=jax.ShapeDtypeStruct((N_NODES, D_FEAT), jnp.float32),
        grid=(N_NODES // _BLK,),
        in_specs=[
            pl.BlockSpec(memory_space=pltpu.SMEM),
            pl.BlockSpec((NC, _BLK, D_FEAT), lambda i: (0, i, 0)),
        ],
        out_specs=pl.BlockSpec((_BLK, D_FEAT), lambda i: (i, 0)),
    )(alpha_arr, parts)


def kernel(x, edge_index, edge_weight, alphas_raw):
    alphas = jnp.tanh(alphas_raw.astype(jnp.float32))
    src = edge_index[0].astype(jnp.int32)
    dst = edge_index[1].astype(jnp.int32)
    w = edge_weight.astype(jnp.float32)

    pad = E_PAD - N_EDGES
    src_p = jnp.pad(src, (0, pad)).reshape(NW, NCH, CHUNK)
    dst_p = jnp.pad(dst, (0, pad)).reshape(NW, NCH, CHUNK)
    w_p = jnp.pad(w, (0, pad)).reshape(NW, NCH, CHUNK)
    zeros = jnp.zeros((N_NODES, D_FEAT), jnp.float32)

    xs = [x]
    cur = x
    for L in range(1, DEPTH + 1):
        parts = _spmm_partial(cur, src_p, dst_p, w_p, zeros)
        cur = _combine(parts, alphas[L])
        xs.append(cur)
    return jnp.stack(xs, axis=1)


# speedup vs baseline: 1.1544x; 1.1544x over previous
import functools

import jax
import jax.numpy as jnp
from jax import lax
from jax.experimental import pallas as pl
from jax.experimental.pallas import tpu as pltpu
from jax.experimental.pallas import tpu_sc as plsc

N_NODES = 10000
D_FEAT = 128
N_EDGES = 320000
DEPTH = 3

NC = 2
NS = 16
NW = NC * NS
CHUNK = 48
NCH = 213
EPT = CHUNK * NCH
E_PAD = EPT * NW
TRIPLES = NCH // 3
RPT = 632
RPT_LAST = N_NODES - (NS - 1) * RPT

_mesh = plsc.VectorSubcoreMesh(
    core_axis_name="c", subcore_axis_name="s", num_cores=NC, num_subcores=NS
)


@functools.partial(
    pl.kernel,
    out_type=jax.ShapeDtypeStruct((NC, N_NODES, D_FEAT), jnp.float32),
    mesh=_mesh,
    scratch_types=[
        pltpu.VMEM_SHARED((N_NODES, D_FEAT), jnp.float32),
        pltpu.VMEM((NCH, CHUNK), jnp.int32),
        pltpu.VMEM((NCH, CHUNK), jnp.int32),
        pltpu.VMEM((CHUNK,), jnp.float32),
        pltpu.VMEM((CHUNK,), jnp.float32),
        pltpu.VMEM((CHUNK,), jnp.float32),
        pltpu.VMEM((CHUNK, D_FEAT), jnp.bfloat16),
        pltpu.VMEM((CHUNK, D_FEAT), jnp.bfloat16),
        pltpu.VMEM((CHUNK, D_FEAT), jnp.bfloat16),
        pltpu.VMEM((CHUNK, D_FEAT), jnp.float32),
        pltpu.VMEM((CHUNK, D_FEAT), jnp.float32),
        pltpu.VMEM((CHUNK, D_FEAT), jnp.float32),
        pltpu.SemaphoreType.DMA,
        pltpu.SemaphoreType.DMA,
        pltpu.SemaphoreType.DMA,
        pltpu.SemaphoreType.DMA,
        pltpu.SemaphoreType.DMA,
        pltpu.SemaphoreType.DMA,
        pltpu.SemaphoreType.DMA,
        pltpu.SemaphoreType.DMA,
        pltpu.SemaphoreType.DMA,
    ],
    compiler_params=pltpu.CompilerParams(use_tc_tiling_on_sc=False,
                                         needs_layout_passes=False),
)
def _spmm_partial(x_hbm, src_hbm, dst_hbm, w_hbm, zeros_hbm, part_hbm,
                  acc, src_all, dst_all, wb0, wb1, wb2, rb0, rb1, rb2,
                  fb0, fb1, fb2, sw0, sw1, sw2, sg0, sg1, sg2, ss0, ss1, ss2):
    c = lax.axis_index("c")
    s = lax.axis_index("s")
    wid = c * NS + s
    rbase = pl.multiple_of(s * RPT, 8)

    @pl.when(s < NS - 1)
    def _zero_main():
        pltpu.sync_copy(zeros_hbm.at[pl.ds(rbase, RPT)], acc.at[pl.ds(rbase, RPT)])

    @pl.when(s == NS - 1)
    def _zero_last():
        pltpu.sync_copy(zeros_hbm.at[pl.ds(rbase, RPT_LAST)],
                        acc.at[pl.ds(rbase, RPT_LAST)])

    pltpu.sync_copy(src_hbm.at[wid], src_all)
    pltpu.sync_copy(dst_hbm.at[wid], dst_all)

    plsc.subcore_barrier()

    wbufs = (wb0, wb1, wb2)
    rows_bufs = (rb0, rb1, rb2)
    frows_bufs = (fb0, fb1, fb2)
    sw = (sw0, sw1, sw2)
    sg = (sg0, sg1, sg2)
    ss = (ss0, ss1, ss2)

    def start_fetch(k, b):
        pltpu.async_copy(w_hbm.at[wid, k], wbufs[b], sw[b])
        pltpu.async_copy(x_hbm.at[src_all.at[k]], rows_bufs[b], sg[b])

    def wait_fetch(k, b):
        pltpu.make_async_copy(w_hbm.at[wid, k], wbufs[b], sw[b]).wait()
        pltpu.make_async_copy(x_hbm.at[src_all.at[k]], rows_bufs[b], sg[b]).wait()

    def mul_rows(k, b):
        rowsb = rows_bufs[b]
        frowsb = frows_bufs[b]

        def group(g, carry):
            wv = wbufs[b][pl.ds(g * 16, 16)]
            for e in range(16):
                row = g * 16 + e
                wgt = wv[e]
                for j in range(D_FEAT // 32):
                    pa, pb = plsc.unpack(rowsb[row, pl.ds(j * 32, 32)],
                                         format=plsc.PackFormat.INTERLEAVED)
                    frowsb[row, pl.ds(j * 32, 16)] = pa * wgt
                    frowsb[row, pl.ds(j * 32 + 16, 16)] = pb * wgt
            return carry

        lax.fori_loop(0, CHUNK // 16, group, 0)

    def start_scatter(k, b):
        pltpu.async_copy(frows_bufs[b], acc.at[dst_all.at[k]], ss[b], add=True)

    def wait_scatter(k, b):
        pltpu.make_async_copy(frows_bufs[b], acc.at[dst_all.at[k]], ss[b]).wait()

    start_fetch(0, 0)
    start_fetch(1, 1)

    def triple_body(t, carry):
        for u in range(3):
            k = 3 * t + u
            wait_fetch(k, u)
            mul_rows(k, u)
            start_scatter(k, u)
            b2 = (u + 2) % 3
            if u == 0:
                @pl.when(t > 0)
                def _refill0():
                    wait_scatter(k - 1, b2)
                    start_fetch(k + 2, b2)

                @pl.when(t == 0)
                def _first_fill():
                    start_fetch(k + 2, b2)
            else:
                @pl.when(t < TRIPLES - 1)
                def _refill():
                    wait_scatter(k - 1, b2)
                    start_fetch(k + 2, b2)
        return carry

    lax.fori_loop(0, TRIPLES, triple_body, 0)

    wait_scatter(NCH - 3, (NCH - 3) % 3)
    wait_scatter(NCH - 2, (NCH - 2) % 3)
    wait_scatter(NCH - 1, (NCH - 1) % 3)

    plsc.subcore_barrier()

    @pl.when(s < NS - 1)
    def _out_main():
        pltpu.sync_copy(acc.at[pl.ds(rbase, RPT)],
                        part_hbm.at[c, pl.ds(rbase, RPT)])

    @pl.when(s == NS - 1)
    def _out_last():
        pltpu.sync_copy(acc.at[pl.ds(rbase, RPT_LAST)],
                        part_hbm.at[c, pl.ds(rbase, RPT_LAST)])


def _combine_body(alpha_ref, p_ref, o_ref):
    o_ref[...] = alpha_ref[0] * (p_ref[0] + p_ref[1])


_BLK = 1000


def _combine(parts, alpha):
    alpha_arr = jnp.reshape(alpha, (1,)).astype(jnp.float32)
    return pl.pallas_call(
        _combine_body,
        out_shape=jax.ShapeDtypeStruct((N_NODES, D_FEAT), jnp.float32),
        grid=(N_NODES // _BLK,),
        in_specs=[
            pl.BlockSpec(memory_space=pltpu.SMEM),
            pl.BlockSpec((NC, _BLK, D_FEAT), lambda i: (0, i, 0)),
        ],
        out_specs=pl.BlockSpec((_BLK, D_FEAT), lambda i: (i, 0)),
    )(alpha_arr, parts)


def kernel(x, edge_index, edge_weight, alphas_raw):
    alphas = jnp.tanh(alphas_raw.astype(jnp.float32))
    src = edge_index[0].astype(jnp.int32)
    dst = edge_index[1].astype(jnp.int32)
    w = edge_weight.astype(jnp.float32)

    pad = E_PAD - N_EDGES
    src_p = jnp.pad(src, (0, pad)).reshape(NW, NCH, CHUNK)
    dst_p = jnp.pad(dst, (0, pad)).reshape(NW, NCH, CHUNK)
    w_p = jnp.pad(w, (0, pad)).reshape(NW, NCH, CHUNK)
    zeros = jnp.zeros((N_NODES, D_FEAT), jnp.float32)

    inv = []
    for f in range(D_FEAT):
        j, r = f // 32, f % 32
        inv.append(j * 32 + (r // 2 if r % 2 == 0 else 16 + (r - 1) // 2))
    inv = jnp.array(inv, jnp.int32)

    xs = [x]
    cur = x
    for L in range(1, DEPTH + 1):
        parts = _spmm_partial(cur.astype(jnp.bfloat16), src_p, dst_p, w_p, zeros)
        cur = _combine(parts, alphas[L])[:, inv]
        xs.append(cur)
    return jnp.stack(xs, axis=1)


# speedup vs baseline: 1.8215x; 1.5778x over previous
import functools

import jax
import jax.numpy as jnp
from jax import lax
from jax.experimental import pallas as pl
from jax.experimental.pallas import tpu as pltpu
from jax.experimental.pallas import tpu_sc as plsc

N_NODES = 10000
D_FEAT = 128
N_EDGES = 320000
DEPTH = 3

NC = 2
NS = 16
DH = D_FEAT // NC
CHUNK = 96
NCH = 210
EPT = CHUNK * NCH
E_PAD = EPT * NS
TRIPLES = NCH // 3
RPT = 632
RPT_LAST = N_NODES - (NS - 1) * RPT

_mesh = plsc.VectorSubcoreMesh(
    core_axis_name="c", subcore_axis_name="s", num_cores=NC, num_subcores=NS
)


@functools.partial(
    pl.kernel,
    out_type=jax.ShapeDtypeStruct((NC, N_NODES, DH), jnp.float32),
    mesh=_mesh,
    scratch_types=[
        pltpu.VMEM_SHARED((N_NODES, DH), jnp.float32),
        pltpu.VMEM((NCH, CHUNK), jnp.int32),
        pltpu.VMEM((NCH, CHUNK), jnp.int32),
        pltpu.VMEM((NCH, CHUNK), jnp.float32),
        pltpu.VMEM((CHUNK,), jnp.int32),
        pltpu.VMEM((CHUNK,), jnp.int32),
        pltpu.VMEM((CHUNK,), jnp.int32),
        pltpu.VMEM((CHUNK, DH), jnp.bfloat16),
        pltpu.VMEM((CHUNK, DH), jnp.bfloat16),
        pltpu.VMEM((CHUNK, DH), jnp.bfloat16),
        pltpu.VMEM((CHUNK, DH), jnp.float32),
        pltpu.VMEM((CHUNK, DH), jnp.float32),
        pltpu.VMEM((CHUNK, DH), jnp.float32),
        pltpu.VMEM((32,), jnp.int32),
        pltpu.VMEM((16,), jnp.float32),
        pltpu.SemaphoreType.DMA,
        pltpu.SemaphoreType.DMA,
        pltpu.SemaphoreType.DMA,
        pltpu.SemaphoreType.DMA,
        pltpu.SemaphoreType.DMA,
        pltpu.SemaphoreType.DMA,
    ],
    compiler_params=pltpu.CompilerParams(use_tc_tiling_on_sc=False,
                                         needs_layout_passes=False),
)
def _spmm_fsplit(xview_hbm, src_hbm, dst_hbm, w_hbm, consts_hbm, alpha_hbm,
                 zeros_hbm, y_hbm,
                 acc, src_all, dst_all, w_all,
                 idx0, idx1, idx2, rows0, rows1, rows2, frows0, frows1, frows2,
                 consts_v, alpha_v,
                 sg0, sg1, sg2, ss0, ss1, ss2):
    c = lax.axis_index("c")
    s = lax.axis_index("s")
    rbase = pl.multiple_of(s * RPT, 8)

    @pl.when(s < NS - 1)
    def _zero_main():
        pltpu.sync_copy(zeros_hbm.at[pl.ds(rbase, RPT)], acc.at[pl.ds(rbase, RPT)])

    @pl.when(s == NS - 1)
    def _zero_last():
        pltpu.sync_copy(zeros_hbm.at[pl.ds(rbase, RPT_LAST)],
                        acc.at[pl.ds(rbase, RPT_LAST)])

    pltpu.sync_copy(src_hbm.at[s], src_all)
    pltpu.sync_copy(dst_hbm.at[s], dst_all)
    pltpu.sync_copy(w_hbm.at[s], w_all)
    pltpu.sync_copy(consts_hbm, consts_v)
    pltpu.sync_copy(alpha_hbm, alpha_v)

    av = consts_v[pl.ds(0, 16)]
    bv = consts_v[pl.ds(16, 16)]
    cb = c * bv
    alpha = alpha_v[...]

    plsc.subcore_barrier()

    idx_bufs = (idx0, idx1, idx2)
    rows_bufs = (rows0, rows1, rows2)
    frows_bufs = (frows0, frows1, frows2)
    sg = (sg0, sg1, sg2)
    ss = (ss0, ss1, ss2)

    def start_gather(k, b):
        idxb = idx_bufs[b]
        for g in range(CHUNK // 16):
            sl = pl.ds(g * 16, 16)
            idxb[sl] = src_all[k, sl] * av + cb
        pltpu.async_copy(xview_hbm.at[idxb], rows_bufs[b], sg[b])

    def wait_gather(b):
        pltpu.make_async_copy(xview_hbm.at[idx_bufs[b]], rows_bufs[b], sg[b]).wait()

    def mul_rows(k, b):
        rowsb = rows_bufs[b]
        frowsb = frows_bufs[b]
        for g in range(CHUNK // 16):
            wv = w_all[k, pl.ds(g * 16, 16)] * alpha
            for e in range(16):
                row = g * 16 + e
                wgt = wv[e]
                for j in range(DH // 32):
                    pa, pb = plsc.unpack(rowsb[row, pl.ds(j * 32, 32)],
                                         format=plsc.PackFormat.INTERLEAVED)
                    frowsb[row, pl.ds(j * 32, 16)] = pa * wgt
                    frowsb[row, pl.ds(j * 32 + 16, 16)] = pb * wgt

    def start_scatter(k, b):
        pltpu.async_copy(frows_bufs[b], acc.at[dst_all.at[k]], ss[b], add=True)

    def wait_scatter(k, b):
        pltpu.make_async_copy(frows_bufs[b], acc.at[dst_all.at[k]], ss[b]).wait()

    start_gather(0, 0)
    start_gather(1, 1)

    def triple_body(t, carry):
        for u in range(3):
            k = 3 * t + u
            wait_gather(u)
            mul_rows(k, u)
            start_scatter(k, u)
            b2 = (u + 2) % 3

            if u == 0:
                @pl.when(t > 0)
                def _refill0():
                    wait_scatter(k - 1, b2)
                    start_gather(k + 2, b2)

                @pl.when(t == 0)
                def _first_fill():
                    start_gather(k + 2, b2)
            elif u == 1:
                @pl.when(t < TRIPLES - 1)
                def _refill1():
                    wait_scatter(k - 1, b2)
                    start_gather(k + 2, b2)
            else:
                @pl.when(t < TRIPLES - 1)
                def _refill2():
                    wait_scatter(k - 1, b2)
                    start_gather(k + 2, b2)
        return carry

    lax.fori_loop(0, TRIPLES, triple_body, 0)

    wait_scatter(NCH - 3, (NCH - 3) % 3)
    wait_scatter(NCH - 2, (NCH - 2) % 3)
    wait_scatter(NCH - 1, (NCH - 1) % 3)

    plsc.subcore_barrier()

    @pl.when(s < NS - 1)
    def _out_main():
        pltpu.sync_copy(acc.at[pl.ds(rbase, RPT)],
                        y_hbm.at[c, pl.ds(rbase, RPT)])

    @pl.when(s == NS - 1)
    def _out_last():
        pltpu.sync_copy(acc.at[pl.ds(rbase, RPT_LAST)],
                        y_hbm.at[c, pl.ds(rbase, RPT_LAST)])


def kernel(x, edge_index, edge_weight, alphas_raw):
    alphas = jnp.tanh(alphas_raw.astype(jnp.float32))
    src = edge_index[0].astype(jnp.int32)
    dst = edge_index[1].astype(jnp.int32)
    w = edge_weight.astype(jnp.float32)

    pad = E_PAD - N_EDGES
    src_p = jnp.pad(src, (0, pad)).reshape(NS, NCH, CHUNK)
    dst_p = jnp.pad(dst, (0, pad)).reshape(NS, NCH, CHUNK)
    w_p = jnp.pad(w, (0, pad)).reshape(NS, NCH, CHUNK)
    zeros = jnp.zeros((N_NODES, DH), jnp.float32)

    consts_x = jnp.concatenate([jnp.full((16,), NC, jnp.int32),
                                jnp.full((16,), 1, jnp.int32)])
    consts_y = jnp.concatenate([jnp.full((16,), 1, jnp.int32),
                                jnp.full((16,), N_NODES, jnp.int32)])

    inv = []
    for f in range(DH):
        j, r = f // 32, f % 32
        inv.append(j * 32 + (r // 2 if r % 2 == 0 else 16 + (r - 1) // 2))
    inv = jnp.array(inv, jnp.int32)

    xs = [x]
    xview = x.astype(jnp.bfloat16).reshape(NC * N_NODES, DH)
    consts = consts_x
    for L in range(1, DEPTH + 1):
        alpha_vec = jnp.full((16,), 1.0, jnp.float32) * alphas[L]
        y = _spmm_fsplit(xview, src_p, dst_p, w_p, consts, alpha_vec, zeros)
        y = y[:, :, inv]
        xs.append(y.transpose(1, 0, 2).reshape(N_NODES, D_FEAT))
        xview = y.astype(jnp.bfloat16).reshape(NC * N_NODES, DH)
        consts = consts_y
    return jnp.stack(xs, axis=1)


# speedup vs baseline: 1.8875x; 1.0363x over previous
import functools

import jax
import jax.numpy as jnp
from jax import lax
from jax.experimental import pallas as pl
from jax.experimental.pallas import tpu as pltpu
from jax.experimental.pallas import tpu_sc as plsc

N_NODES = 10000
D_FEAT = 128
N_EDGES = 320000
DEPTH = 3

NC = 2
NS = 16
DH = D_FEAT // NC
CHUNK = 96
NCH = 210
EPT = CHUNK * NCH
E_PAD = EPT * NS
TRIPLES = NCH // 3
RPT = 632
RPT_LAST = N_NODES - (NS - 1) * RPT

_mesh = plsc.VectorSubcoreMesh(
    core_axis_name="c", subcore_axis_name="s", num_cores=NC, num_subcores=NS
)


@functools.partial(
    pl.kernel,
    out_type=jax.ShapeDtypeStruct((NC, N_NODES, DH), jnp.float32),
    mesh=_mesh,
    scratch_types=[
        pltpu.VMEM_SHARED((N_NODES, DH), jnp.float32),
        pltpu.VMEM_SHARED((N_NODES, DH), jnp.bfloat16),
        pltpu.VMEM((NCH, CHUNK), jnp.int32),
        pltpu.VMEM((NCH, CHUNK), jnp.int32),
        pltpu.VMEM((CHUNK,), jnp.float32),
        pltpu.VMEM((CHUNK,), jnp.float32),
        pltpu.VMEM((CHUNK,), jnp.float32),
        pltpu.VMEM((CHUNK,), jnp.int32),
        pltpu.VMEM((CHUNK,), jnp.int32),
        pltpu.VMEM((CHUNK,), jnp.int32),
        pltpu.VMEM((CHUNK, DH), jnp.bfloat16),
        pltpu.VMEM((CHUNK, DH), jnp.bfloat16),
        pltpu.VMEM((CHUNK, DH), jnp.bfloat16),
        pltpu.VMEM((CHUNK, DH), jnp.float32),
        pltpu.VMEM((CHUNK, DH), jnp.float32),
        pltpu.VMEM((CHUNK, DH), jnp.float32),
        pltpu.VMEM((32,), jnp.int32),
        pltpu.VMEM((16,), jnp.float32),
        pltpu.SemaphoreType.DMA,
        pltpu.SemaphoreType.DMA,
        pltpu.SemaphoreType.DMA,
        pltpu.SemaphoreType.DMA,
        pltpu.SemaphoreType.DMA,
        pltpu.SemaphoreType.DMA,
        pltpu.SemaphoreType.DMA,
        pltpu.SemaphoreType.DMA,
        pltpu.SemaphoreType.DMA,
    ],
    compiler_params=pltpu.CompilerParams(use_tc_tiling_on_sc=False,
                                         needs_layout_passes=False),
)
def _spmm_fsplit(xview_hbm, src_hbm, dst_hbm, w_hbm, consts_hbm, alpha_hbm,
                 zeros_hbm, y_hbm,
                 acc, table, src_all, dst_all, wb0, wb1, wb2,
                 idx0, idx1, idx2, rows0, rows1, rows2, frows0, frows1, frows2,
                 consts_v, alpha_v,
                 sw0, sw1, sw2, sg0, sg1, sg2, ss0, ss1, ss2):
    c = lax.axis_index("c")
    s = lax.axis_index("s")
    rbase = pl.multiple_of(s * RPT, 8)

    @pl.when(s < NS - 1)
    def _zero_main():
        pltpu.sync_copy(zeros_hbm.at[pl.ds(rbase, RPT)], acc.at[pl.ds(rbase, RPT)])

    @pl.when(s == NS - 1)
    def _zero_last():
        pltpu.sync_copy(zeros_hbm.at[pl.ds(rbase, RPT_LAST)],
                        acc.at[pl.ds(rbase, RPT_LAST)])

    pltpu.sync_copy(src_hbm.at[s], src_all)
    pltpu.sync_copy(dst_hbm.at[s], dst_all)
    pltpu.sync_copy(consts_hbm, consts_v)
    pltpu.sync_copy(alpha_hbm, alpha_v)
    tb = pl.multiple_of(c * N_NODES + rbase, 8)

    @pl.when(s < NS - 1)
    def _tab_main():
        pltpu.sync_copy(xview_hbm.at[pl.ds(tb, RPT)], table.at[pl.ds(rbase, RPT)])

    @pl.when(s == NS - 1)
    def _tab_last():
        pltpu.sync_copy(xview_hbm.at[pl.ds(tb, RPT_LAST)],
                        table.at[pl.ds(rbase, RPT_LAST)])

    av = consts_v[pl.ds(0, 16)]
    bv = consts_v[pl.ds(16, 16)]
    cb = c * bv
    alpha = alpha_v[...]

    plsc.subcore_barrier()

    idx_bufs = (idx0, idx1, idx2)
    rows_bufs = (rows0, rows1, rows2)
    frows_bufs = (frows0, frows1, frows2)
    wbufs = (wb0, wb1, wb2)
    sw = (sw0, sw1, sw2)
    sg = (sg0, sg1, sg2)
    ss = (ss0, ss1, ss2)

    def start_gather(k, b, spmem):
        pltpu.async_copy(w_hbm.at[s, k], wbufs[b], sw[b])
        if spmem:
            pltpu.async_copy(table.at[src_all.at[k]], rows_bufs[b], sg[b])
        else:
            idxb = idx_bufs[b]
            for g in range(CHUNK // 16):
                sl = pl.ds(g * 16, 16)
                idxb[sl] = src_all[k, sl] * av + cb
            pltpu.async_copy(xview_hbm.at[idxb], rows_bufs[b], sg[b])

    def wait_gather(k, b, spmem):
        pltpu.make_async_copy(w_hbm.at[s, k], wbufs[b], sw[b]).wait()
        if spmem:
            pltpu.make_async_copy(table.at[src_all.at[k]],
                                  rows_bufs[b], sg[b]).wait()
        else:
            pltpu.make_async_copy(xview_hbm.at[idx_bufs[b]],
                                  rows_bufs[b], sg[b]).wait()

    def mul_rows(k, b):
        rowsb = rows_bufs[b]
        frowsb = frows_bufs[b]
        for g in range(CHUNK // 16):
            wv = wbufs[b][pl.ds(g * 16, 16)] * alpha
            for e in range(16):
                row = g * 16 + e
                wgt = wv[e]
                for j in range(DH // 32):
                    pa, pb = plsc.unpack(rowsb[row, pl.ds(j * 32, 32)],
                                         format=plsc.PackFormat.INTERLEAVED)
                    frowsb[row, pl.ds(j * 32, 16)] = pa * wgt
                    frowsb[row, pl.ds(j * 32 + 16, 16)] = pb * wgt

    def start_scatter(k, b):
        pltpu.async_copy(frows_bufs[b], acc.at[dst_all.at[k]], ss[b], add=True)

    def wait_scatter(k, b):
        pltpu.make_async_copy(frows_bufs[b], acc.at[dst_all.at[k]], ss[b]).wait()

    start_gather(0, 0, False)
    start_gather(1, 1, False)

    def triple_body(t, carry):
        for u in range(3):
            k = 3 * t + u
            spmem_u = (u == 2)
            wait_gather(k, u, spmem_u)
            mul_rows(k, u)
            start_scatter(k, u)
            b2 = (u + 2) % 3
            spmem_b2 = (b2 == 2)

            if u == 0:
                @pl.when(t > 0)
                def _refill0():
                    wait_scatter(k - 1, b2)
                    start_gather(k + 2, b2, spmem_b2)

                @pl.when(t == 0)
                def _first_fill():
                    start_gather(k + 2, b2, spmem_b2)
            elif u == 1:
                @pl.when(t < TRIPLES - 1)
                def _refill1():
                    wait_scatter(k - 1, b2)
                    start_gather(k + 2, b2, spmem_b2)
            else:
                @pl.when(t < TRIPLES - 1)
                def _refill2():
                    wait_scatter(k - 1, b2)
                    start_gather(k + 2, b2, spmem_b2)
        return carry

    lax.fori_loop(0, TRIPLES, triple_body, 0)

    wait_scatter(NCH - 3, (NCH - 3) % 3)
    wait_scatter(NCH - 2, (NCH - 2) % 3)
    wait_scatter(NCH - 1, (NCH - 1) % 3)

    plsc.subcore_barrier()

    @pl.when(s < NS - 1)
    def _out_main():
        pltpu.sync_copy(acc.at[pl.ds(rbase, RPT)],
                        y_hbm.at[c, pl.ds(rbase, RPT)])

    @pl.when(s == NS - 1)
    def _out_last():
        pltpu.sync_copy(acc.at[pl.ds(rbase, RPT_LAST)],
                        y_hbm.at[c, pl.ds(rbase, RPT_LAST)])


def kernel(x, edge_index, edge_weight, alphas_raw):
    alphas = jnp.tanh(alphas_raw.astype(jnp.float32))
    src = edge_index[0].astype(jnp.int32)
    dst = edge_index[1].astype(jnp.int32)
    w = edge_weight.astype(jnp.float32)

    pad = E_PAD - N_EDGES
    src_p = jnp.pad(src, (0, pad)).reshape(NS, NCH, CHUNK)
    dst_p = jnp.pad(dst, (0, pad)).reshape(NS, NCH, CHUNK)
    w_p = jnp.pad(w, (0, pad)).reshape(NS, NCH, CHUNK)
    zeros = jnp.zeros((N_NODES, DH), jnp.float32)

    consts_x = jnp.concatenate([jnp.full((16,), NC, jnp.int32),
                                jnp.full((16,), 1, jnp.int32)])
    consts_y = jnp.concatenate([jnp.full((16,), 1, jnp.int32),
                                jnp.full((16,), N_NODES, jnp.int32)])

    inv = []
    for f in range(DH):
        j, r = f // 32, f % 32
        inv.append(j * 32 + (r // 2 if r % 2 == 0 else 16 + (r - 1) // 2))
    inv = jnp.array(inv, jnp.int32)

    xs = [x]
    xview = (x.astype(jnp.bfloat16).reshape(N_NODES, NC, DH)
             .transpose(1, 0, 2).reshape(NC * N_NODES, DH))
    consts = consts_y
    for L in range(1, DEPTH + 1):
        alpha_vec = jnp.full((16,), 1.0, jnp.float32) * alphas[L]
        y = _spmm_fsplit(xview, src_p, dst_p, w_p, consts, alpha_vec, zeros)
        y = y[:, :, inv]
        xs.append(y.transpose(1, 0, 2).reshape(N_NODES, D_FEAT))
        xview = y.astype(jnp.bfloat16).reshape(NC * N_NODES, DH)
        consts = consts_y
    return jnp.stack(xs, axis=1)
